# baseline (speedup 1.0000x reference)
import functools

import jax
import jax.numpy as jnp
from jax import lax
from jax.experimental import pallas as pl
from jax.experimental.pallas import tpu as pltpu
from jax.experimental.pallas import tpu_sc as plsc

VOCAB = 1_000_000
D = 64
NSEQS = 16384
SEQ = 50
NTOK = NSEQS * SEQ
IDXW = 128
IDXROWS = NTOK // IDXW

NC, NS = 2, 16
NW = NC * NS

K = 4
NBUF = 2

ROWS_PER_W = IDXROWS // NW
NSTEPS = ROWS_PER_W // K

SBLK = 128

_mesh = plsc.VectorSubcoreMesh(
    core_axis_name="c", subcore_axis_name="s", num_cores=NC, num_subcores=NS
)


@functools.partial(
    pl.kernel,
    out_type=jax.ShapeDtypeStruct((NTOK, D), jnp.float32),
    mesh=_mesh,
    scratch_types=[
        pltpu.VMEM((NBUF, K, IDXW), jnp.int32),
        pltpu.VMEM((NBUF, K * IDXW, D), jnp.float32),
        pltpu.SemaphoreType.DMA,
        pltpu.SemaphoreType.DMA,
    ],
    compiler_params=pltpu.CompilerParams(use_tc_tiling_on_sc=False),
)
def _embed_gather(table_hbm, idx_hbm, out_hbm, idx_v, rows_v, sem0, sem1):
    sems = (sem0, sem1)
    wid = lax.axis_index("s") * NC + lax.axis_index("c")
    row0 = wid * ROWS_PER_W

    def fire(slot, s):
        pltpu.sync_copy(idx_hbm.at[pl.ds(row0 + s * K, K)], idx_v.at[slot])
        for j in range(K):
            pltpu.async_copy(
                table_hbm.at[idx_v.at[slot, j]],
                rows_v.at[slot, pl.ds(j * IDXW, IDXW)],
                sems[slot],
            )

    def drain_flush(slot, s):
        pltpu.make_async_copy(
            out_hbm.at[pl.ds(0, K * IDXW)],
            rows_v.at[slot],
            sems[slot],
        ).wait()
        pltpu.sync_copy(
            rows_v.at[slot], out_hbm.at[pl.ds((row0 + s * K) * IDXW, K * IDXW)]
        )

    for b in range(NBUF):
        fire(b, b)

    @pl.loop(0, NSTEPS, step=NBUF)
    def _(g):
        for b in range(NBUF):
            s = g + b
            drain_flush(b, s)

            @pl.when(s + NBUF < NSTEPS)
            def _():
                fire(b, s + NBUF)


def _retile_body(i_ref, o_ref):
    x = i_ref[...]
    x3 = x.reshape(SBLK, 25, IDXW)
    t3 = jnp.transpose(x3, (1, 2, 0))
    t4 = t3.reshape(25, 2, D, SBLK)
    t5 = t4.reshape(SEQ, D, SBLK)
    o_ref[...] = t5.reshape(SEQ, D // 8, 1, 8, SBLK)


_retile = pl.pallas_call(
    _retile_body,
    out_shape=jax.ShapeDtypeStruct((SEQ, D // 8, NSEQS // SBLK, 8, SBLK), jnp.float32),
    grid=(NSEQS // SBLK,),
    in_specs=[
        pl.BlockSpec((NTOK * D // IDXW // (NSEQS // SBLK), IDXW), lambda g: (g, 0))
    ],
    out_specs=pl.BlockSpec(
        (SEQ, D // 8, 1, 8, SBLK), lambda g: (0, 0, g, 0, 0)
    ),
)


def kernel(token_ids, embeddings):
    idx = token_ids.astype(jnp.int32).reshape(IDXROWS, IDXW)
    flat = _embed_gather(embeddings, idx)
    out5 = _retile(flat.reshape(NTOK * D // IDXW, IDXW))
    return jnp.transpose(out5, (2, 4, 0, 1, 3)).reshape(NSEQS, SEQ, D)


# speedup vs baseline: 1.0648x; 1.0648x over previous
import functools

import jax
import jax.numpy as jnp
from jax import lax
from jax.experimental import pallas as pl
from jax.experimental.pallas import tpu as pltpu
from jax.experimental.pallas import tpu_sc as plsc

VOCAB = 1_000_000
D = 64
NSEQS = 16384
SEQ = 50
NTOK = NSEQS * SEQ
IDXW = 128
IDXROWS = NTOK // IDXW

NCHUNK = 4
CSEQS = NSEQS // NCHUNK
CTOK = CSEQS * SEQ
CROWS = CTOK // IDXW

NC, NS = 2, 16
NW = NC * NS

K = 5
NBUF = 2

ROWS_PER_W = CROWS // NW
NSTEPS = ROWS_PER_W // K

SBLK = 128
CTILES = CSEQS // SBLK
TCB = 1

_mesh = plsc.VectorSubcoreMesh(
    core_axis_name="c", subcore_axis_name="s", num_cores=NC, num_subcores=NS
)


def _make_gather(chunk):
    @functools.partial(
        pl.kernel,
        out_type=jax.ShapeDtypeStruct((CTOK, D), jnp.float32),
        mesh=_mesh,
        scratch_types=[
            pltpu.VMEM((NBUF, K, IDXW), jnp.int32),
            pltpu.VMEM((NBUF, K * IDXW, D), jnp.float32),
            pltpu.SemaphoreType.DMA,
            pltpu.SemaphoreType.DMA,
        ],
        compiler_params=pltpu.CompilerParams(use_tc_tiling_on_sc=False),
    )
    def _embed_gather(table_hbm, idx_hbm, out_hbm, idx_v, rows_v, sem0, sem1):
        sems = (sem0, sem1)
        wid = lax.axis_index("s") * NC + lax.axis_index("c")
        row0 = chunk * CROWS + wid * ROWS_PER_W
        out0 = wid * ROWS_PER_W * IDXW

        def fire(slot, s):
            pltpu.sync_copy(idx_hbm.at[pl.ds(row0 + s * K, K)], idx_v.at[slot])
            for j in range(K):
                pltpu.async_copy(
                    table_hbm.at[idx_v.at[slot, j]],
                    rows_v.at[slot, pl.ds(j * IDXW, IDXW)],
                    sems[slot],
                )

        def drain_flush(slot, s):
            pltpu.make_async_copy(
                out_hbm.at[pl.ds(0, K * IDXW)],
                rows_v.at[slot],
                sems[slot],
            ).wait()
            pltpu.sync_copy(
                rows_v.at[slot],
                out_hbm.at[pl.ds(out0 + s * K * IDXW, K * IDXW)],
            )

        for b in range(NBUF):
            fire(b, b)

        @pl.loop(0, NSTEPS, step=NBUF)
        def _(g):
            for b in range(NBUF):
                s = g + b
                drain_flush(b, s)

                @pl.when(s + NBUF < NSTEPS)
                def _():
                    fire(b, s + NBUF)

    return _embed_gather


def _retile_body(i_ref, o_ref):
    x = i_ref[...]
    x3 = x.reshape(TCB * SBLK, 25, IDXW)
    t3 = jnp.transpose(x3, (1, 2, 0))
    t4 = t3.reshape(25, 2, D // 8, 8, TCB, SBLK)
    t5 = jnp.transpose(t4, (0, 1, 2, 4, 3, 5))
    o_ref[...] = t5.reshape(SEQ, D // 8, TCB, 8, SBLK)


def _aliased_body(i_ref, a_ref, o_ref):
    del a_ref
    _retile_body(i_ref, o_ref)


_O5 = jax.ShapeDtypeStruct((SEQ, D // 8, NSEQS // SBLK, 8, SBLK), jnp.float32)


def _make_retile(chunk, aliased):
    body = _aliased_body if aliased else _retile_body
    in_specs = [
        pl.BlockSpec((TCB * SBLK * 25, IDXW), lambda g: (g, 0)),
    ]
    if aliased:
        in_specs.append(pl.BlockSpec(memory_space=pltpu.MemorySpace.HBM))
    return pl.pallas_call(
        body,
        out_shape=_O5,
        grid=(CTILES // TCB,),
        in_specs=in_specs,
        out_specs=pl.BlockSpec(
            (SEQ, D // 8, TCB, 8, SBLK),
            lambda g: (0, 0, chunk * (CTILES // TCB) + g, 0, 0),
        ),
        input_output_aliases={1: 0} if aliased else {},
    )


_gathers = [_make_gather(h) for h in range(NCHUNK)]
_retiles = [_make_retile(h, h > 0) for h in range(NCHUNK)]


def kernel(token_ids, embeddings):
    idx = token_ids.astype(jnp.int32).reshape(IDXROWS, IDXW)
    out5 = None
    for h in range(NCHUNK):
        flat = _gathers[h](embeddings, idx)
        flat2 = flat.reshape(CTOK * D // IDXW, IDXW)
        if h == 0:
            out5 = _retiles[h](flat2)
        else:
            out5 = _retiles[h](flat2, out5)
    return jnp.transpose(out5, (2, 4, 0, 1, 3)).reshape(NSEQS, SEQ, D)


# speedup vs baseline: 1.0651x; 1.0002x over previous
import functools

import jax
import jax.numpy as jnp
from jax import lax
from jax.experimental import pallas as pl
from jax.experimental.pallas import tpu as pltpu
from jax.experimental.pallas import tpu_sc as plsc

VOCAB = 1_000_000
D = 64
NSEQS = 16384
SEQ = 50
NTOK = NSEQS * SEQ
IDXW = 128
IDXROWS = NTOK // IDXW

NCHUNK = 4
CSEQS = NSEQS // NCHUNK
CTOK = CSEQS * SEQ
CROWS = CTOK // IDXW

NC, NS = 2, 16
NW = NC * NS

K = 5
NBUF = 2

ROWS_PER_W = CROWS // NW
NSTEPS = ROWS_PER_W // K

SBLK = 128
CTILES = CSEQS // SBLK
TCB = 1

_mesh = plsc.VectorSubcoreMesh(
    core_axis_name="c", subcore_axis_name="s", num_cores=NC, num_subcores=NS
)


def _make_gather(chunk):
    @functools.partial(
        pl.kernel,
        out_type=jax.ShapeDtypeStruct((CTOK, D), jnp.float32),
        mesh=_mesh,
        scratch_types=[
            pltpu.VMEM((NBUF, K, IDXW), jnp.int32),
            pltpu.VMEM((NBUF, K * IDXW, D), jnp.float32),
            pltpu.SemaphoreType.DMA,
            pltpu.SemaphoreType.DMA,
        ],
        compiler_params=pltpu.CompilerParams(use_tc_tiling_on_sc=False),
    )
    def _embed_gather(table_hbm, idx_hbm, out_hbm, idx_v, rows_v, sem0, sem1):
        sems = (sem0, sem1)
        wid = lax.axis_index("s") * NC + lax.axis_index("c")
        row0 = chunk * CROWS + wid * ROWS_PER_W
        out0 = wid * ROWS_PER_W * IDXW

        def fire(slot, s):
            pltpu.sync_copy(idx_hbm.at[pl.ds(row0 + s * K, K)], idx_v.at[slot])
            for j in range(K):
                pltpu.async_copy(
                    table_hbm.at[idx_v.at[slot, j]],
                    rows_v.at[slot, pl.ds(j * IDXW, IDXW)],
                    sems[slot],
                )

        def drain_flush(slot, s):
            pltpu.make_async_copy(
                out_hbm.at[pl.ds(0, K * IDXW)],
                rows_v.at[slot],
                sems[slot],
            ).wait()
            pltpu.sync_copy(
                rows_v.at[slot],
                out_hbm.at[pl.ds(out0 + s * K * IDXW, K * IDXW)],
            )

        for b in range(NBUF):
            fire(b, b)

        @pl.loop(0, NSTEPS, step=NBUF)
        def _(g):
            for b in range(NBUF):
                s = g + b
                drain_flush(b, s)

                @pl.when(s + NBUF < NSTEPS)
                def _():
                    fire(b, s + NBUF)

    return _embed_gather


def _retile_body(i_ref, o_ref):
    x = i_ref[...]
    x3 = x.reshape(TCB * SBLK, 25, IDXW)
    t3 = jnp.transpose(x3, (1, 2, 0))
    t4 = t3.reshape(25, 2, D // 8, 8, TCB, SBLK)
    t5 = jnp.transpose(t4, (0, 1, 2, 4, 3, 5))
    o_ref[...] = t5.reshape(SEQ, D // 8, TCB, 8, SBLK)


def _aliased_body(i_ref, a_ref, o_ref):
    del a_ref
    _retile_body(i_ref, o_ref)


_O5 = jax.ShapeDtypeStruct((SEQ, D // 8, NSEQS // SBLK, 8, SBLK), jnp.float32)


def _make_retile(chunk, aliased):
    body = _aliased_body if aliased else _retile_body
    in_specs = [
        pl.BlockSpec((TCB * SBLK * 25, IDXW), lambda g: (g, 0)),
    ]
    if aliased:
        in_specs.append(pl.BlockSpec(memory_space=pltpu.MemorySpace.HBM))
    return pl.pallas_call(
        body,
        out_shape=_O5,
        grid=(CTILES // TCB,),
        in_specs=in_specs,
        out_specs=pl.BlockSpec(
            (SEQ, D // 8, TCB, 8, SBLK),
            lambda g: (0, 0, chunk * (CTILES // TCB) + g, 0, 0),
        ),
        input_output_aliases={1: 0} if aliased else {},
        compiler_params=pltpu.CompilerParams(
            dimension_semantics=("parallel",)
        ),
    )


_gathers = [_make_gather(h) for h in range(NCHUNK)]
_retiles = [_make_retile(h, h > 0) for h in range(NCHUNK)]


def kernel(token_ids, embeddings):
    idx = token_ids.astype(jnp.int32).reshape(IDXROWS, IDXW)
    out5 = None
    for h in range(NCHUNK):
        flat = _gathers[h](embeddings, idx)
        flat2 = flat.reshape(CTOK * D // IDXW, IDXW)
        if h == 0:
            out5 = _retiles[h](flat2)
        else:
            out5 = _retiles[h](flat2, out5)
    return jnp.transpose(out5, (2, 4, 0, 1, 3)).reshape(NSEQS, SEQ, D)
